# baseline (speedup 1.0000x reference)
import functools

import jax
import jax.numpy as jnp
from jax import lax
from jax.experimental import pallas as pl
from jax.experimental.pallas import tpu as pltpu
from jax.experimental.pallas import tpu_sc as plsc

T = 8
B = 16384
E = 64
V = 100000
SV = 1000
L = 16
NC = 2
NS = 16
NW = NC * NS
CHUNK = 128
NCH = (B // NW) // CHUNK
NG = CHUNK // L
VB = 2048
NVB = (V + VB - 1) // VB
PV = NVB * (VB // 2)


def _pack_body(i_ref, o_ref):
    t = i_ref[0].T
    o_ref[0] = jnp.concatenate([t[:VB // 2], t[VB // 2:]], axis=1)


_pack_value = functools.partial(
    pl.pallas_call,
    grid=(T, NVB),
    in_specs=[pl.BlockSpec((1, E, VB), lambda t, v: (t, 0, v))],
    out_specs=pl.BlockSpec((1, VB // 2, 2 * E), lambda t, v: (t, v, 0)),
    out_shape=jax.ShapeDtypeStruct((T, PV, 2 * E), jnp.float32),
)(_pack_body)


def _sc_body(vals_hbm, srcs_hbm, vt_hbm, st_hbm, w_hbm, out_hbm,
             idx_v, idx_s, ridx_v, rows_v, rows_s, acc, obuf, wvec, sem):
    wid = lax.axis_index("s") * NC + lax.axis_index("c")
    base = wid * NCH
    iota = lax.iota(jnp.int32, L)
    for t in range(T):
        pltpu.sync_copy(vals_hbm.at[t].at[pl.ds(base, NCH)], idx_v)
        pltpu.sync_copy(srcs_hbm.at[t].at[pl.ds(base, NCH)], idx_s)
        for j in range(NCH):
            for g in range(NG):
                sl = pl.ds(g * L, L)
                v = idx_v[j, sl]
                ridx_v[j, sl] = (
                    lax.shift_left(lax.shift_right_logical(v, 11), 10)
                    + (v & 1023))
        pltpu.sync_copy(w_hbm.at[t], wvec)
        wv = wvec[...]
        for j in range(NCH):
            cv = pltpu.async_copy(vt_hbm.at[t].at[ridx_v.at[j]], rows_v, sem)
            cs = pltpu.async_copy(st_hbm.at[t].at[idx_s.at[j]], rows_s, sem)
            cv.wait()
            cs.wait()
            rowids = []
            hvs = []
            for g in range(NG):
                sl = pl.ds(g * L, L)
                rowids.append(iota + (g * L))
                hvs.append((lax.shift_right_logical(idx_v[j, sl], 10) & 1) * 64)

            def e_body(e, _, j=j, rowids=rowids, hvs=hvs, wv=wv, t=t):
                esp = lax.broadcast_in_dim(e, (L,), ())
                for g in range(NG):
                    v = plsc.load_gather(rows_v, [rowids[g], hvs[g] + esp])
                    s = plsc.load_gather(rows_s, [rowids[g], esp])
                    prod = v * s * wv
                    if t == 0:
                        acc[j, e, pl.ds(g * L, L)] = prod
                    else:
                        plsc.addupdate(acc.at[j, e, pl.ds(g * L, L)], prod)
                return 0

            lax.fori_loop(0, E, e_body, 0)
    for j in range(NCH):
        jsp = jnp.full((L,), j, dtype=jnp.int32)

        def tr_body(r, _, j=j, jsp=jsp):
            rsp = lax.broadcast_in_dim(r, (L,), ())
            for eb in range(E // L):
                ev = iota + (eb * L)
                obuf[r, pl.ds(eb * L, L)] = plsc.load_gather(acc, [jsp, ev, rsp])
            return 0

        lax.fori_loop(0, CHUNK, tr_body, 0)
        pltpu.sync_copy(obuf, out_hbm.at[base + j])


_sc_embed = functools.partial(
    pl.kernel,
    out_type=jax.ShapeDtypeStruct((NW * NCH, CHUNK, E), jnp.float32),
    mesh=plsc.VectorSubcoreMesh(
        core_axis_name="c", subcore_axis_name="s",
        num_cores=NC, num_subcores=NS),
    scratch_types=[
        pltpu.VMEM((NCH, CHUNK), jnp.int32),
        pltpu.VMEM((NCH, CHUNK), jnp.int32),
        pltpu.VMEM((NCH, CHUNK), jnp.int32),
        pltpu.VMEM((CHUNK, 2 * E), jnp.float32),
        pltpu.VMEM((CHUNK, E), jnp.float32),
        pltpu.VMEM((NCH, E, CHUNK), jnp.float32),
        pltpu.VMEM((CHUNK, E), jnp.float32),
        pltpu.VMEM((L,), jnp.float32),
        pltpu.SemaphoreType.DMA,
    ],
    compiler_params=pltpu.CompilerParams(
        use_tc_tiling_on_sc=False, needs_layout_passes=False),
)(_sc_body)


def kernel(win_values, win_tokens_size, win_sources, win_src_tokens_size,
           value_tables, source_tables, win_weight):
    del win_tokens_size, win_src_tokens_size
    vals = win_values.astype(jnp.int32).reshape(T, NW * NCH, CHUNK)
    srcs = win_sources.astype(jnp.int32).reshape(T, NW * NCH, CHUNK)
    vt = _pack_value(value_tables.transpose(0, 2, 1))
    wexp = jnp.broadcast_to(win_weight[:, None], (T, L))
    out = _sc_embed(vals, srcs, vt, source_tables, wexp)
    return out.reshape(B, E)


# speedup vs baseline: 1.2785x; 1.2785x over previous
import functools

import jax
import jax.numpy as jnp
from jax import lax
from jax.experimental import pallas as pl
from jax.experimental.pallas import tpu as pltpu
from jax.experimental.pallas import tpu_sc as plsc

T = 8
B = 16384
E = 64
V = 100000
SV = 1000
L = 16
NC = 2
NS = 16
NW = NC * NS
CHUNK = 128
NCH = (B // NW) // CHUNK
VB = 2048
NVB = (V + VB - 1) // VB
PV = NVB * (VB // 2)


def _pack_body(i_ref, o_ref):
    blk = i_ref[0]
    ident = jnp.eye(E, dtype=jnp.float32)
    t = lax.dot_general(blk, ident, (((0,), (0,)), ((), ())),
                        precision=lax.Precision.HIGHEST)
    o_ref[0] = jnp.concatenate([t[:VB // 2], t[VB // 2:]], axis=1)


_pack_value = functools.partial(
    pl.pallas_call,
    grid=(T, NVB),
    in_specs=[pl.BlockSpec((1, E, VB), lambda t, v: (t, 0, v))],
    out_specs=pl.BlockSpec((1, VB // 2, 2 * E), lambda t, v: (t, v, 0)),
    out_shape=jax.ShapeDtypeStruct((T, PV, 2 * E), jnp.float32),
)(_pack_body)


def _sc_body(vals_hbm, srcs_hbm, vt_hbm, st_hbm, w_hbm, out_hbm,
             idx_v, idx_s, ridx_v, rows_v, rows_s, acc, wvec,
             sem_v0, sem_s0, sem_v1, sem_s1):
    wid = lax.axis_index("s") * NC + lax.axis_index("c")
    base = wid * NCH
    sems = ((sem_v0, sem_s0), (sem_v1, sem_s1))

    def fire(t, j):
        sv, ss = sems[j % 2]
        cv = pltpu.async_copy(vt_hbm.at[t].at[ridx_v.at[j]], rows_v.at[j % 2], sv)
        cs = pltpu.async_copy(st_hbm.at[t].at[idx_s.at[j]], rows_s.at[j % 2], ss)
        return cv, cs

    for t in range(T):
        pltpu.sync_copy(vals_hbm.at[t].at[pl.ds(base, NCH)], idx_v)
        pltpu.sync_copy(srcs_hbm.at[t].at[pl.ds(base, NCH)], idx_s)
        for j in range(NCH):
            for g in range(CHUNK // L):
                sl = pl.ds(g * L, L)
                v = idx_v[j, sl]
                ridx_v[j, sl] = (
                    lax.shift_left(lax.shift_right_logical(v, 11), 10)
                    + (v & 1023))
        pltpu.sync_copy(w_hbm.at[t], wvec)
        wv = wvec[...]
        pend = fire(t, 0)
        for j in range(NCH):
            nxt = fire(t, j + 1) if j + 1 < NCH else None
            pend[0].wait()
            pend[1].wait()
            pend = nxt
            slot = j % 2

            def row_body(i, _, j=j, slot=slot, wv=wv, t=t):
                idx16 = idx_v[j, pl.ds(i & ~(L - 1), L)]
                sp = lax.broadcast_in_dim(i & (L - 1), (L,), ())
                pv = idx16.at[sp].get(mode="promise_in_bounds")
                sel = (lax.shift_right_logical(pv, 10) & 1) != 0
                for e in range(0, E, L):
                    lo = rows_v[slot, i, pl.ds(e, L)]
                    hi = rows_v[slot, i, pl.ds(E + e, L)]
                    s = rows_s[slot, i, pl.ds(e, L)]
                    prod = jnp.where(sel, hi, lo) * s * wv
                    if t == 0:
                        acc[j, i, pl.ds(e, L)] = prod
                    else:
                        plsc.addupdate(acc.at[j, i, pl.ds(e, L)], prod)
                return 0

            lax.fori_loop(0, CHUNK, row_body, 0)
    for j in range(NCH):
        pltpu.sync_copy(acc.at[j], out_hbm.at[base + j])


_sc_embed = functools.partial(
    pl.kernel,
    out_type=jax.ShapeDtypeStruct((NW * NCH, CHUNK, E), jnp.float32),
    mesh=plsc.VectorSubcoreMesh(
        core_axis_name="c", subcore_axis_name="s",
        num_cores=NC, num_subcores=NS),
    scratch_types=[
        pltpu.VMEM((NCH, CHUNK), jnp.int32),
        pltpu.VMEM((NCH, CHUNK), jnp.int32),
        pltpu.VMEM((NCH, CHUNK), jnp.int32),
        pltpu.VMEM((2, CHUNK, 2 * E), jnp.float32),
        pltpu.VMEM((2, CHUNK, E), jnp.float32),
        pltpu.VMEM((NCH, CHUNK, E), jnp.float32),
        pltpu.VMEM((L,), jnp.float32),
        pltpu.SemaphoreType.DMA,
        pltpu.SemaphoreType.DMA,
        pltpu.SemaphoreType.DMA,
        pltpu.SemaphoreType.DMA,
    ],
    compiler_params=pltpu.CompilerParams(
        use_tc_tiling_on_sc=False, needs_layout_passes=False),
)(_sc_body)


def kernel(win_values, win_tokens_size, win_sources, win_src_tokens_size,
           value_tables, source_tables, win_weight):
    del win_tokens_size, win_src_tokens_size
    vals = win_values.astype(jnp.int32).reshape(T, NW * NCH, CHUNK)
    srcs = win_sources.astype(jnp.int32).reshape(T, NW * NCH, CHUNK)
    vt = _pack_value(value_tables.transpose(0, 2, 1))
    wexp = jnp.broadcast_to(win_weight[:, None], (T, L))
    out = _sc_embed(vals, srcs, vt, source_tables, wexp)
    return out.reshape(B, E)


# speedup vs baseline: 1.5427x; 1.2066x over previous
import functools

import jax
import jax.numpy as jnp
from jax import lax
from jax.experimental import pallas as pl
from jax.experimental.pallas import tpu as pltpu
from jax.experimental.pallas import tpu_sc as plsc

T = 8
B = 16384
E = 64
V = 100000
SV = 1000
L = 16
NC = 2
NS = 16
NW = NC * NS
H0 = 50176
H1 = V - H0
IC = 4096
NICH = B // IC
NG = IC // L


def _pack_idx_body(v_ref, s_ref, o_ref):
    o_ref[...] = v_ref[...] * 1024 + s_ref[...]


_pack_idx = functools.partial(
    pl.pallas_call,
    out_shape=jax.ShapeDtypeStruct((T, B), jnp.int32),
)(_pack_idx_body)


def _sc_body(cidx_hbm, vt_hbm, st_hbm, w_hbm, out_hbm,
             vbufA, vbufB, cbuf, acc, srow, wvec,
             sem_a, sem_b, sem_c0, sem_c1):
    wid = lax.axis_index("s") * NC + lax.axis_index("c")
    csems = (sem_c0, sem_c1)
    zero = jnp.zeros((L,), jnp.float32)

    def half_a_src(t, e):
        return vt_hbm.at[t].at[e].at[pl.ds(0, H0)]

    def do_pass(half, t, wv):
        vb = vbufA if half == 0 else vbufB
        off = 0 if half == 0 else H0
        pltpu.async_copy(cidx_hbm.at[t].at[pl.ds(0, IC)], cbuf.at[0], sem_c0)
        for c in range(NICH):
            if c + 1 < NICH:
                pltpu.async_copy(
                    cidx_hbm.at[t].at[pl.ds((c + 1) * IC, IC)],
                    cbuf.at[(c + 1) % 2], csems[(c + 1) % 2])
            pltpu.make_async_copy(
                cidx_hbm.at[t].at[pl.ds(c * IC, IC)],
                cbuf.at[c % 2], csems[c % 2]).wait()
            slot = c % 2

            def grp(g, _, c=c, slot=slot, half=half, off=off, wv=wv, vb=vb):
                ci = cbuf[slot, pl.ds(g * L, L)]
                v = lax.shift_right_logical(ci, 10)
                s = ci & 1023
                if half == 0:
                    m = v < H0
                else:
                    m = v >= H0
                vloc = jnp.where(m, v - off, 0)
                vg = plsc.load_gather(vb, [vloc], mask=m)
                sg = plsc.load_gather(srow, [s])
                prod = jnp.where(m, vg, 0.0) * sg * wv
                plsc.addupdate(
                    acc.at[c * 32 + g // 8, pl.ds((g % 8) * L, L)], prod)
                return 0

            lax.fori_loop(0, NG, grp, 0)

    for k in range(2):
        e = wid * 2 + k

        def _zero(i, _):
            acc[i // 8, pl.ds((i % 8) * L, L)] = zero
            return 0

        lax.fori_loop(0, B // L, _zero, 0)

        if k == 0:
            pltpu.async_copy(half_a_src(0, e), vbufA, sem_a)

        def t_body(t, _, k=k, e=e):
            pltpu.sync_copy(st_hbm.at[t].at[e], srow)
            pltpu.sync_copy(w_hbm.at[t], wvec)
            wv = wvec[...]
            pltpu.async_copy(vt_hbm.at[t].at[e].at[pl.ds(H0, H1)],
                             vbufB, sem_b)
            pltpu.make_async_copy(half_a_src(t, e), vbufA, sem_a).wait()
            do_pass(0, t, wv)
            if k == 0:
                nt = jnp.where(t + 1 < T, t + 1, 0)
                ne = jnp.where(t + 1 < T, e, e + 1)
                pltpu.async_copy(half_a_src(nt, ne), vbufA, sem_a)
            else:
                @pl.when(t + 1 < T)
                def _():
                    pltpu.async_copy(
                        half_a_src(jnp.minimum(t + 1, T - 1), e),
                        vbufA, sem_a)
            pltpu.make_async_copy(
                vt_hbm.at[t].at[e].at[pl.ds(H0, H1)], vbufB, sem_b).wait()
            do_pass(1, t, wv)
            return 0

        lax.fori_loop(0, T, t_body, 0)
        pltpu.sync_copy(acc, out_hbm.at[e])


_sc_embed = functools.partial(
    pl.kernel,
    out_type=jax.ShapeDtypeStruct((E, B // 128, 128), jnp.float32),
    mesh=plsc.VectorSubcoreMesh(
        core_axis_name="c", subcore_axis_name="s",
        num_cores=NC, num_subcores=NS),
    scratch_types=[
        pltpu.VMEM((H0,), jnp.float32),
        pltpu.VMEM((H1,), jnp.float32),
        pltpu.VMEM((2, IC), jnp.int32),
        pltpu.VMEM((B // 128, 128), jnp.float32),
        pltpu.VMEM((SV,), jnp.float32),
        pltpu.VMEM((L,), jnp.float32),
        pltpu.SemaphoreType.DMA,
        pltpu.SemaphoreType.DMA,
        pltpu.SemaphoreType.DMA,
        pltpu.SemaphoreType.DMA,
    ],
    compiler_params=pltpu.CompilerParams(
        use_tc_tiling_on_sc=True, needs_layout_passes=False),
)(_sc_body)


def kernel(win_values, win_tokens_size, win_sources, win_src_tokens_size,
           value_tables, source_tables, win_weight):
    del win_tokens_size, win_src_tokens_size
    cidx = _pack_idx(win_values.astype(jnp.int32), win_sources)
    wexp = jnp.broadcast_to(win_weight[:, None], (T, L))
    out_t = _sc_embed(cidx, value_tables.transpose(0, 2, 1),
                      source_tables.transpose(0, 2, 1), wexp)
    return out_t.reshape(E, B).T
